# baseline (speedup 1.0000x reference)
import jax
import jax.numpy as jnp
from jax import lax
from jax.experimental import pallas as pl
from jax.experimental.pallas import tpu as pltpu
from jax.experimental.pallas import tpu_sc as plsc

N = 36
D = 8
NV = (N * D) // 16
INV_DEG = 1.0 / float(N - 1)

_f32 = jnp.float32
_i32 = jnp.int32

_DNUMS = lax.GatherDimensionNumbers(
    offset_dims=(), collapsed_slice_dims=(0,), start_index_map=(0,))


def _tk(v, idx):
  return lax.gather(v, idx[:, None], dimension_numbers=_DNUMS,
                    slice_sizes=(1,),
                    mode=lax.GatherScatterMode.PROMISE_IN_BOUNDS)


def _body(x_hbm, wl_hbm, b_hbm, wr_hbm, out_hbm, x_v, wl_v, wr_v, b_v, out_v,
          acol_v, sem):
  c = lax.axis_index("c")
  s = lax.axis_index("s")

  @pl.when(jnp.logical_and(c == 0, s == 0))
  def _():
    copies = [
        pltpu.async_copy(x_hbm, x_v, sem),
        pltpu.async_copy(wl_hbm, wl_v, sem),
        pltpu.async_copy(wr_hbm, wr_v, sem),
        pltpu.async_copy(b_hbm, b_v.at[pl.ds(0, 8)], sem),
    ]
    for cp in copies:
      cp.wait()

    iota = lax.iota(_i32, 16)
    m8 = iota % 8
    half8 = iota - m8
    zero = iota - iota

    sel_lo = [m8 == 2 * sv for sv in range(4)]
    sel_hi = [m8 == 2 * sv + 1 for sv in range(4)]
    sel_or = [jnp.logical_or(sel_lo[sv], sel_hi[sv]) for sv in range(4)]

    wl = [wl_v[pl.ds(16 * m, 16)] for m in range(4)]
    wr = [wr_v[pl.ds(16 * m, 16)] for m in range(4)]
    b2 = _tk(b_v[...], m8)

    av = [wr[m] - wl[m] * INV_DEG for m in range(4)]

    def sum_body(t, tot):
      return tot + x_v[pl.ds(pl.multiple_of(16 * t, 16), 16)]
    tot = lax.fori_loop(1, NV, sum_body, x_v[pl.ds(0, 16)])
    s8 = tot + _tk(tot, m8 + 8)
    s_dup = _tk(s8, m8)

    def wcol_body(k, rv):
      idx = [(jnp.where(sel_lo[sv], k, 0) + jnp.where(sel_hi[sv], k + 8, 0))
             for sv in range(4)]
      ac = _tk(av[3], idx[3])
      wc = _tk(wl[3], idx[3])
      for sv in (2, 1, 0):
        ac = jnp.where(sel_or[sv], _tk(av[sv], idx[sv]), ac)
        wc = jnp.where(sel_or[sv], _tk(wl[sv], idx[sv]), wc)
      acol_v[pl.ds(pl.multiple_of(16 * k, 16), 16)] = ac
      sk = _tk(s_dup, zero + k)
      return rv + sk * wc * INV_DEG
    rv = lax.fori_loop(0, D, wcol_body, b2)

    def row_body(t, carry):
      base = pl.multiple_of(16 * t, 16)
      xt = x_v[pl.ds(base, 16)]
      acc = rv
      for k in range(D):
        xb = _tk(xt, half8 + k)
        acc = acc + xb * acol_v[pl.ds(16 * k, 16)]
      out_v[pl.ds(base, 16)] = acc
      return carry
    lax.fori_loop(0, NV, row_body, 0)

    pltpu.sync_copy(out_v, out_hbm)


_sc_call = pl.kernel(
    _body,
    out_type=jax.ShapeDtypeStruct((N * D,), _f32),
    mesh=plsc.VectorSubcoreMesh(core_axis_name="c", subcore_axis_name="s",
                                num_cores=1),
    scratch_types=[
        pltpu.VMEM((N * D,), _f32),
        pltpu.VMEM((D * D,), _f32),
        pltpu.VMEM((D * D,), _f32),
        pltpu.VMEM((16,), _f32),
        pltpu.VMEM((N * D,), _f32),
        pltpu.VMEM((D * 16,), _f32),
        pltpu.SemaphoreType.DMA,
    ],
)


def kernel(x, W_l, b_l, W_r):
  out_flat = _sc_call(x.reshape(N * D), W_l.reshape(D * D), b_l,
                      W_r.reshape(D * D))
  return out_flat.reshape(N, D)


# speedup vs baseline: 1.0077x; 1.0077x over previous
import jax
import jax.numpy as jnp
from jax import lax
from jax.experimental import pallas as pl
from jax.experimental.pallas import tpu as pltpu
from jax.experimental.pallas import tpu_sc as plsc

N = 36
D = 8
NV = (N * D) // 16
INV_DEG = 1.0 / float(N - 1)

_f32 = jnp.float32
_i32 = jnp.int32

_DNUMS = lax.GatherDimensionNumbers(
    offset_dims=(), collapsed_slice_dims=(0,), start_index_map=(0,))


def _tk(v, idx):
  return lax.gather(v, idx[:, None], dimension_numbers=_DNUMS,
                    slice_sizes=(1,),
                    mode=lax.GatherScatterMode.PROMISE_IN_BOUNDS)


def _body(x_hbm, wl_hbm, b_hbm, wr_hbm, out_hbm, x_v, wl_v, wr_v, b_v, out_v,
          sem, xsem):
  c = lax.axis_index("c")
  s = lax.axis_index("s")

  @pl.when(jnp.logical_and(c == 0, s == 0))
  def _():
    x_copy = pltpu.async_copy(x_hbm, x_v, xsem)
    copies = [
        pltpu.async_copy(wl_hbm, wl_v, sem),
        pltpu.async_copy(wr_hbm, wr_v, sem),
        pltpu.async_copy(b_hbm, b_v.at[pl.ds(0, 8)], sem),
    ]
    for cp in copies:
      cp.wait()

    iota = lax.iota(_i32, 16)
    m8 = iota % 8
    half8 = iota - m8
    zero = iota - iota

    sel_lo = [m8 == 2 * sv for sv in range(4)]
    sel_hi = [m8 == 2 * sv + 1 for sv in range(4)]
    sel_or = [jnp.logical_or(sel_lo[sv], sel_hi[sv]) for sv in range(4)]
    colidx = {}
    for sv in range(4):
      for k in range(D):
        colidx[(sv, k)] = (jnp.where(sel_lo[sv], k, 0)
                           + jnp.where(sel_hi[sv], k + 8, 0))

    def col_dup(wv, k):
      res = _tk(wv[3], colidx[(3, k)])
      for sv in (2, 1, 0):
        res = jnp.where(sel_or[sv], _tk(wv[sv], colidx[(sv, k)]), res)
      return res

    wl = [wl_v[pl.ds(16 * m, 16)] for m in range(4)]
    wr = [wr_v[pl.ds(16 * m, 16)] for m in range(4)]
    b2 = _tk(b_v[...], m8)

    av = [wr[m] - wl[m] * INV_DEG for m in range(4)]
    a_col = [col_dup(av, k) for k in range(D)]
    wl_col = [col_dup(wl, k) for k in range(D)]

    x_copy.wait()

    def sum_body(t, tot):
      return tot + x_v[pl.ds(pl.multiple_of(16 * t, 16), 16)]
    tot = lax.fori_loop(1, NV, sum_body, x_v[pl.ds(0, 16)])
    s8 = tot + _tk(tot, m8 + 8)
    s_dup = _tk(s8, m8)

    rv = b2
    for k in range(D):
      sk = _tk(s_dup, zero + k)
      rv = rv + sk * wl_col[k] * INV_DEG

    xbidx = [half8 + k for k in range(D)]

    def row_body(t, carry):
      base = pl.multiple_of(16 * t, 16)
      xt = x_v[pl.ds(base, 16)]
      acc = rv
      for k in range(D):
        acc = acc + _tk(xt, xbidx[k]) * a_col[k]
      out_v[pl.ds(base, 16)] = acc
      return carry
    lax.fori_loop(0, NV, row_body, 0)

    pltpu.sync_copy(out_v, out_hbm)


_sc_call = pl.kernel(
    _body,
    out_type=jax.ShapeDtypeStruct((N * D,), _f32),
    mesh=plsc.VectorSubcoreMesh(core_axis_name="c", subcore_axis_name="s",
                                num_cores=1),
    scratch_types=[
        pltpu.VMEM((N * D,), _f32),
        pltpu.VMEM((D * D,), _f32),
        pltpu.VMEM((D * D,), _f32),
        pltpu.VMEM((16,), _f32),
        pltpu.VMEM((N * D,), _f32),
        pltpu.SemaphoreType.DMA,
        pltpu.SemaphoreType.DMA,
    ],
)


def kernel(x, W_l, b_l, W_r):
  out_flat = _sc_call(x.reshape(N * D), W_l.reshape(D * D), b_l,
                      W_r.reshape(D * D))
  return out_flat.reshape(N, D)
